# Initial kernel scaffold; baseline (speedup 1.0000x reference)
import jax
import jax.numpy as jnp
from jax.experimental import pallas as pl


def kernel(x, edge_index, W1, b1, W2, b2):
    raise NotImplementedError("write your pallas kernel here")


# speedup vs baseline: 33.8883x; 33.8883x over previous
import functools

import jax
import jax.numpy as jnp
from jax import lax
from jax.experimental import pallas as pl
from jax.experimental.pallas import tpu as pltpu
from jax.experimental.pallas import tpu_sc as plsc

N = 10000
E = 320000
F_IN = 128
HID = 16
C = 40

NC = 2
NS = 16
W = NC * NS
EP = E // W
CH = 125
NCH = EP // CH
RP = N // NS
DEG_PAD = 10240
DRP = DEG_PAD // NS


def _mesh():
    return plsc.VectorSubcoreMesh(core_axis_name="c", subcore_axis_name="s")


_SC_PARAMS = pltpu.CompilerParams(use_tc_tiling_on_sc=False)


def _deg_body(dstr, ones_h, zerosd, out, dst_idx, ones_v, dacc, _sem):
    c = lax.axis_index("c")
    s = lax.axis_index("s")
    w = c * NS + s
    pltpu.sync_copy(dstr.at[w], dst_idx)
    pltpu.sync_copy(ones_h, ones_v)
    pltpu.sync_copy(zerosd, dacc.at[pl.ds(s * DRP, DRP)])
    plsc.subcore_barrier()

    def body(j, carry):
        pltpu.sync_copy(ones_v, dacc.at[dst_idx.at[j]], add=True)
        return carry

    lax.fori_loop(0, NCH, body, 0)
    plsc.subcore_barrier()
    pltpu.sync_copy(dacc.at[pl.ds(s * DRP, DRP)], out.at[c, pl.ds(s * DRP, DRP)])


def _sc_degree(dstr, ones_h, zerosd):
    k = pl.kernel(
        _deg_body,
        out_type=jax.ShapeDtypeStruct((NC, DEG_PAD), jnp.float32),
        mesh=_mesh(),
        compiler_params=_SC_PARAMS,
        scratch_types=[
            pltpu.VMEM((NCH, CH), jnp.int32),
            pltpu.VMEM((CH,), jnp.float32),
            pltpu.VMEM_SHARED((DEG_PAD,), jnp.float32),
            pltpu.SemaphoreType.DMA,
        ],
    )
    return k(dstr, ones_h, zerosd)


def _agg_body(table, srcr, dstr, zeros16, out, src_idx, dst_idx, rows, acc, gsem):
    c = lax.axis_index("c")
    s = lax.axis_index("s")
    w = c * NS + s
    pltpu.sync_copy(srcr.at[w], src_idx)
    pltpu.sync_copy(dstr.at[w], dst_idx)
    pltpu.sync_copy(zeros16, acc.at[pl.ds(s * RP, RP)])
    plsc.subcore_barrier()

    def body(j, carry):
        pltpu.async_copy(table.at[src_idx.at[j]], rows, gsem).wait()
        pltpu.sync_copy(rows, acc.at[dst_idx.at[j]], add=True)
        return carry

    lax.fori_loop(0, NCH, body, 0)
    plsc.subcore_barrier()
    pltpu.sync_copy(acc.at[pl.ds(s * RP, RP)], out.at[c, pl.ds(s * RP, RP)])


def _sc_aggregate(table, srcr, dstr, zeros16):
    k = pl.kernel(
        _agg_body,
        out_type=jax.ShapeDtypeStruct((NC, N, HID), jnp.float32),
        mesh=_mesh(),
        compiler_params=_SC_PARAMS,
        scratch_types=[
            pltpu.VMEM((NCH, CH), jnp.int32),
            pltpu.VMEM((NCH, CH), jnp.int32),
            pltpu.VMEM((CH, HID), jnp.float32),
            pltpu.VMEM_SHARED((N, HID), jnp.float32),
            pltpu.SemaphoreType.DMA,
        ],
    )
    return k(table, srcr, dstr, zeros16)


def _tc_a_body(x_ref, w1_ref, degt_ref, t1_ref, dinv_ref):
    mm = jnp.dot(x_ref[...], w1_ref[...], preferred_element_type=jnp.float32)
    d = degt_ref[:, 0:1] + degt_ref[:, 1:2] + 1.0
    dinv = lax.rsqrt(d)
    t1_ref[...] = mm * dinv
    dinv_ref[...] = dinv


def _tc_a(x, w1, degt):
    return pl.pallas_call(
        _tc_a_body,
        out_shape=(
            jax.ShapeDtypeStruct((N, HID), jnp.float32),
            jax.ShapeDtypeStruct((N, 1), jnp.float32),
        ),
    )(x, w1, degt)


def _tc_b_body(a0_ref, a1_ref, t1_ref, dinv_ref, b1_ref, t2_ref):
    dinv = dinv_ref[...]
    z = dinv * (a0_ref[...] + a1_ref[...] + t1_ref[...]) + b1_ref[...]
    t2_ref[...] = dinv * jnp.maximum(z, 0.0)


def _tc_b(a0, a1, t1, dinv, b1):
    return pl.pallas_call(
        _tc_b_body,
        out_shape=jax.ShapeDtypeStruct((N, HID), jnp.float32),
    )(a0, a1, t1, dinv, b1)


def _tc_c_body(a0_ref, a1_ref, t2_ref, dinv_ref, w2_ref, b2_ref, out_ref):
    u = dinv_ref[...] * (a0_ref[...] + a1_ref[...] + t2_ref[...])
    z = jnp.dot(u, w2_ref[...], preferred_element_type=jnp.float32) + b2_ref[...]
    m = jnp.max(z, axis=1, keepdims=True)
    zs = z - m
    lse = jnp.log(jnp.sum(jnp.exp(zs), axis=1, keepdims=True))
    out_ref[...] = zs - lse


def _tc_c(a0, a1, t2, dinv, w2, b2):
    return pl.pallas_call(
        _tc_c_body,
        out_shape=jax.ShapeDtypeStruct((N, C), jnp.float32),
    )(a0, a1, t2, dinv, w2, b2)


def kernel(x, edge_index, W1, b1, W2, b2):
    srcr = edge_index[0].reshape(W, NCH, CH)
    dstr = edge_index[1].reshape(W, NCH, CH)
    ones_h = jnp.ones((CH,), jnp.float32)
    zerosd = jnp.zeros((DRP,), jnp.float32)
    zeros16 = jnp.zeros((RP, HID), jnp.float32)

    deg2 = _sc_degree(dstr, ones_h, zerosd)
    degt = deg2[:, :N].T
    t1, dinv = _tc_a(x, W1, degt)
    a1 = _sc_aggregate(t1, srcr, dstr, zeros16)
    t2 = _tc_b(a1[0], a1[1], t1, dinv, b1.reshape(1, HID))
    a2 = _sc_aggregate(t2, srcr, dstr, zeros16)
    return _tc_c(a2[0], a2[1], t2, dinv, W2, b2.reshape(1, C))


# speedup vs baseline: 57.6228x; 1.7004x over previous
import functools

import jax
import jax.numpy as jnp
from jax import lax
from jax.experimental import pallas as pl
from jax.experimental.pallas import tpu as pltpu
from jax.experimental.pallas import tpu_sc as plsc

N = 10000
E = 320000
F_IN = 128
HID = 16
C = 40

NC = 2
NS = 16
W = NC * NS
EP = E // W
CH = 125
NCH = EP // CH
NB = 8
RP = N // NS
DEG_PAD = 10240
DRP = DEG_PAD // NS


def _mesh():
    return plsc.VectorSubcoreMesh(core_axis_name="c", subcore_axis_name="s")


_SC_PARAMS = pltpu.CompilerParams(use_tc_tiling_on_sc=False)


def _deg_body(dstr, ones_h, zerosd, out, dst_idx, ones_v, dacc, ssem):
    c = lax.axis_index("c")
    s = lax.axis_index("s")
    w = c * NS + s
    pltpu.sync_copy(dstr.at[w], dst_idx)
    pltpu.sync_copy(ones_h, ones_v)
    pltpu.sync_copy(zerosd, dacc.at[pl.ds(s * DRP, DRP)])
    plsc.subcore_barrier()

    def body(g, carry):
        sd = [
            pltpu.async_copy(ones_v, dacc.at[dst_idx.at[g * NB + b]], ssem, add=True)
            for b in range(NB)
        ]
        for d in sd:
            d.wait()
        return carry

    lax.fori_loop(0, NCH // NB, body, 0)
    plsc.subcore_barrier()
    pltpu.sync_copy(dacc.at[pl.ds(s * DRP, DRP)], out.at[c, pl.ds(s * DRP, DRP)])


def _sc_degree(dstr, ones_h, zerosd):
    k = pl.kernel(
        _deg_body,
        out_type=jax.ShapeDtypeStruct((NC, DEG_PAD), jnp.float32),
        mesh=_mesh(),
        compiler_params=_SC_PARAMS,
        scratch_types=[
            pltpu.VMEM((NCH, CH), jnp.int32),
            pltpu.VMEM((CH,), jnp.float32),
            pltpu.VMEM_SHARED((DEG_PAD,), jnp.float32),
            pltpu.SemaphoreType.DMA,
        ],
    )
    return k(dstr, ones_h, zerosd)


def _agg_body(table, srcr, dstr, zeros16, out, src_idx, dst_idx, rows, acc, gsem, ssem):
    c = lax.axis_index("c")
    s = lax.axis_index("s")
    w = c * NS + s
    pltpu.sync_copy(srcr.at[w], src_idx)
    pltpu.sync_copy(dstr.at[w], dst_idx)
    pltpu.sync_copy(zeros16, acc.at[pl.ds(s * RP, RP)])
    plsc.subcore_barrier()

    def body(g, carry):
        gd = [
            pltpu.async_copy(table.at[src_idx.at[g * NB + b]], rows.at[b], gsem)
            for b in range(NB)
        ]
        sd = []
        for b in range(NB):
            gd[b].wait()
            sd.append(
                pltpu.async_copy(rows.at[b], acc.at[dst_idx.at[g * NB + b]], ssem, add=True)
            )
        for d in sd:
            d.wait()
        return carry

    lax.fori_loop(0, NCH // NB, body, 0)
    plsc.subcore_barrier()
    pltpu.sync_copy(acc.at[pl.ds(s * RP, RP)], out.at[c, pl.ds(s * RP, RP)])


def _sc_aggregate(table, srcr, dstr, zeros16):
    k = pl.kernel(
        _agg_body,
        out_type=jax.ShapeDtypeStruct((NC, N, HID), jnp.float32),
        mesh=_mesh(),
        compiler_params=_SC_PARAMS,
        scratch_types=[
            pltpu.VMEM((NCH, CH), jnp.int32),
            pltpu.VMEM((NCH, CH), jnp.int32),
            pltpu.VMEM((NB, CH, HID), jnp.float32),
            pltpu.VMEM_SHARED((N, HID), jnp.float32),
            pltpu.SemaphoreType.DMA,
            pltpu.SemaphoreType.DMA,
        ],
    )
    return k(table, srcr, dstr, zeros16)


def _tc_a_body(x_ref, w1_ref, degt_ref, t1_ref, dinv_ref):
    mm = jnp.dot(x_ref[...], w1_ref[...], preferred_element_type=jnp.float32)
    d = degt_ref[:, 0:1] + degt_ref[:, 1:2] + 1.0
    dinv = lax.rsqrt(d)
    t1_ref[...] = mm * dinv
    dinv_ref[...] = dinv


def _tc_a(x, w1, degt):
    return pl.pallas_call(
        _tc_a_body,
        out_shape=(
            jax.ShapeDtypeStruct((N, HID), jnp.float32),
            jax.ShapeDtypeStruct((N, 1), jnp.float32),
        ),
    )(x, w1, degt)


def _tc_b_body(a_ref, t1_ref, dinv_ref, b1_ref, t2_ref):
    dinv = dinv_ref[...]
    z = dinv * (a_ref[0] + a_ref[1] + t1_ref[...]) + b1_ref[...]
    t2_ref[...] = dinv * jnp.maximum(z, 0.0)


def _tc_b(a, t1, dinv, b1):
    return pl.pallas_call(
        _tc_b_body,
        out_shape=jax.ShapeDtypeStruct((N, HID), jnp.float32),
    )(a, t1, dinv, b1)


def _tc_c_body(a_ref, t2_ref, dinv_ref, w2_ref, b2_ref, out_ref):
    u = dinv_ref[...] * (a_ref[0] + a_ref[1] + t2_ref[...])
    z = jnp.dot(u, w2_ref[...], preferred_element_type=jnp.float32) + b2_ref[...]
    m = jnp.max(z, axis=1, keepdims=True)
    zs = z - m
    lse = jnp.log(jnp.sum(jnp.exp(zs), axis=1, keepdims=True))
    out_ref[...] = zs - lse


def _tc_c(a, t2, dinv, w2, b2):
    return pl.pallas_call(
        _tc_c_body,
        out_shape=jax.ShapeDtypeStruct((N, C), jnp.float32),
    )(a, t2, dinv, w2, b2)


def kernel(x, edge_index, W1, b1, W2, b2):
    srcr = edge_index[0].reshape(W, NCH, CH)
    dstr = edge_index[1].reshape(W, NCH, CH)
    ones_h = jnp.ones((CH,), jnp.float32)
    zerosd = jnp.zeros((DRP,), jnp.float32)
    zeros16 = jnp.zeros((RP, HID), jnp.float32)

    deg2 = _sc_degree(dstr, ones_h, zerosd)
    degt = deg2[:, :N].T
    t1, dinv = _tc_a(x, W1, degt)
    a1 = _sc_aggregate(t1, srcr, dstr, zeros16)
    t2 = _tc_b(a1, t1, dinv, b1.reshape(1, HID))
    a2 = _sc_aggregate(t2, srcr, dstr, zeros16)
    return _tc_c(a2, t2, dinv, W2, b2.reshape(1, C))


# speedup vs baseline: 58.4780x; 1.0148x over previous
import jax
import jax.numpy as jnp
from jax import lax
from jax.experimental import pallas as pl
from jax.experimental.pallas import tpu as pltpu
from jax.experimental.pallas import tpu_sc as plsc

N = 10000
E = 320000
F_IN = 128
HID = 16
C = 40

NC = 2
NS = 16
W = NC * NS
EPW = E // W
CH = 128
NCH = 80
PADE = NCH * CH - EPW
NB = 8
NP = 10240
RP = NP // NS
DEGC = 2 * NCH


def _mesh():
    return plsc.VectorSubcoreMesh(core_axis_name="c", subcore_axis_name="s")


_SC_PARAMS = pltpu.CompilerParams(
    use_tc_tiling_on_sc=False, needs_layout_passes=False)


def _rsqrt_sc(d):
    bi = plsc.bitcast(d, jnp.int32)
    yi = jnp.int32(0x5F3759DF) - lax.shift_right_logical(bi, 1)
    y = plsc.bitcast(yi, jnp.float32)
    y = y * (1.5 - 0.5 * d * y * y)
    y = y * (1.5 - 0.5 * d * y * y)
    return y


def _mega1_body(mm, eip, ones_h, zerosd, zeros16,
                a1, t1o, dinvo,
                dst_all, src_idx, dst_idx, rows, ones_v, dv, mv,
                t1_sp, acc, dacc, gsem, ssem):
    c = lax.axis_index("c")
    s = lax.axis_index("s")
    w = c * NS + s
    r0 = s * RP

    pltpu.sync_copy(eip.at[1, 2 * s], dst_all.at[pl.ds(0, NCH)])
    pltpu.sync_copy(eip.at[1, 2 * s + 1], dst_all.at[pl.ds(NCH, NCH)])
    pltpu.sync_copy(eip.at[0, w], src_idx)
    pltpu.sync_copy(eip.at[1, w], dst_idx)
    pltpu.sync_copy(ones_h, ones_v)
    pltpu.sync_copy(zerosd, dacc.at[pl.ds(r0, RP)])
    pltpu.sync_copy(zeros16, acc.at[pl.ds(r0, RP)])
    pltpu.sync_copy(mm.at[pl.ds(r0, RP)], mv)
    plsc.subcore_barrier()

    def degloop(g, cy):
        sd = [
            pltpu.async_copy(ones_v, dacc.at[dst_all.at[g * NB + b]], ssem, add=True)
            for b in range(NB)
        ]
        for d in sd:
            d.wait()
        return cy

    lax.fori_loop(0, DEGC // NB, degloop, 0)
    plsc.subcore_barrier()

    pltpu.sync_copy(dacc.at[pl.ds(r0, RP)], dv)

    def dloop(g, cy):
        base = g * 16
        y = _rsqrt_sc(dv[pl.ds(base, 16)] + 1.0)
        dv[pl.ds(base, 16)] = y
        for i in range(16):
            mv[base + i] = mv[base + i] * y[i]
        return cy

    lax.fori_loop(0, RP // 16, dloop, 0)
    pltpu.sync_copy(mv, t1_sp.at[pl.ds(r0, RP)])

    @pl.when(c == 0)
    def _():
        pltpu.sync_copy(mv, t1o.at[pl.ds(r0, RP)])
        pltpu.sync_copy(dv, dinvo.at[pl.ds(r0, RP)])

    plsc.subcore_barrier()

    def aggloop(g, cy):
        gd = [
            pltpu.async_copy(t1_sp.at[src_idx.at[g * NB + b]], rows.at[b], gsem)
            for b in range(NB)
        ]
        sd = []
        for b in range(NB):
            gd[b].wait()
            sd.append(
                pltpu.async_copy(rows.at[b], acc.at[dst_idx.at[g * NB + b]], ssem, add=True)
            )
        for d in sd:
            d.wait()
        return cy

    lax.fori_loop(0, NCH // NB, aggloop, 0)
    plsc.subcore_barrier()
    pltpu.sync_copy(acc.at[pl.ds(r0, RP)], a1.at[c, pl.ds(r0, RP)])


def _mega1(mm, eip, ones_h, zerosd, zeros16):
    k = pl.kernel(
        _mega1_body,
        out_type=(
            jax.ShapeDtypeStruct((NC, NP, HID), jnp.float32),
            jax.ShapeDtypeStruct((NP, HID), jnp.float32),
            jax.ShapeDtypeStruct((NP,), jnp.float32),
        ),
        mesh=_mesh(),
        compiler_params=_SC_PARAMS,
        scratch_types=[
            pltpu.VMEM((DEGC, CH), jnp.int32),
            pltpu.VMEM((NCH, CH), jnp.int32),
            pltpu.VMEM((NCH, CH), jnp.int32),
            pltpu.VMEM((NB, CH, HID), jnp.float32),
            pltpu.VMEM((CH,), jnp.float32),
            pltpu.VMEM((RP,), jnp.float32),
            pltpu.VMEM((RP, HID), jnp.float32),
            pltpu.VMEM_SHARED((NP, HID), jnp.float32),
            pltpu.VMEM_SHARED((NP, HID), jnp.float32),
            pltpu.VMEM_SHARED((NP,), jnp.float32),
            pltpu.SemaphoreType.DMA,
            pltpu.SemaphoreType.DMA,
        ],
    )
    return k(mm, eip, ones_h, zerosd, zeros16)


def _mega2_body(a1, t1o, dinvo, b1h, eip, zeros16,
                a2, t2o,
                src_idx, dst_idx, rows, b1v, dv, p0, p1, t1s,
                t2_sp, acc, gsem, ssem):
    c = lax.axis_index("c")
    s = lax.axis_index("s")
    w = c * NS + s
    r0 = s * RP

    pltpu.sync_copy(eip.at[0, w], src_idx)
    pltpu.sync_copy(eip.at[1, w], dst_idx)
    pltpu.sync_copy(b1h, b1v)
    pltpu.sync_copy(zeros16, acc.at[pl.ds(r0, RP)])
    pltpu.sync_copy(dinvo.at[pl.ds(r0, RP)], dv)
    pltpu.sync_copy(a1.at[0, pl.ds(r0, RP)], p0)
    pltpu.sync_copy(a1.at[1, pl.ds(r0, RP)], p1)
    pltpu.sync_copy(t1o.at[pl.ds(r0, RP)], t1s)
    b1vec = b1v[...]

    def tloop(g, cy):
        base = g * 16
        y = dv[pl.ds(base, 16)]
        for i in range(16):
            yi = y[i]
            z = yi * (p0[base + i] + p1[base + i] + t1s[base + i]) + b1vec
            p0[base + i] = yi * jnp.maximum(z, 0.0)
        return cy

    lax.fori_loop(0, RP // 16, tloop, 0)
    pltpu.sync_copy(p0, t2_sp.at[pl.ds(r0, RP)])

    @pl.when(c == 0)
    def _():
        pltpu.sync_copy(p0, t2o.at[pl.ds(r0, RP)])

    plsc.subcore_barrier()

    def aggloop(g, cy):
        gd = [
            pltpu.async_copy(t2_sp.at[src_idx.at[g * NB + b]], rows.at[b], gsem)
            for b in range(NB)
        ]
        sd = []
        for b in range(NB):
            gd[b].wait()
            sd.append(
                pltpu.async_copy(rows.at[b], acc.at[dst_idx.at[g * NB + b]], ssem, add=True)
            )
        for d in sd:
            d.wait()
        return cy

    lax.fori_loop(0, NCH // NB, aggloop, 0)
    plsc.subcore_barrier()
    pltpu.sync_copy(acc.at[pl.ds(r0, RP)], a2.at[c, pl.ds(r0, RP)])


def _mega2(a1, t1o, dinvo, b1, eip, zeros16):
    k = pl.kernel(
        _mega2_body,
        out_type=(
            jax.ShapeDtypeStruct((NC, NP, HID), jnp.float32),
            jax.ShapeDtypeStruct((NP, HID), jnp.float32),
        ),
        mesh=_mesh(),
        compiler_params=_SC_PARAMS,
        scratch_types=[
            pltpu.VMEM((NCH, CH), jnp.int32),
            pltpu.VMEM((NCH, CH), jnp.int32),
            pltpu.VMEM((NB, CH, HID), jnp.float32),
            pltpu.VMEM((HID,), jnp.float32),
            pltpu.VMEM((RP,), jnp.float32),
            pltpu.VMEM((RP, HID), jnp.float32),
            pltpu.VMEM((RP, HID), jnp.float32),
            pltpu.VMEM((RP, HID), jnp.float32),
            pltpu.VMEM_SHARED((NP, HID), jnp.float32),
            pltpu.VMEM_SHARED((NP, HID), jnp.float32),
            pltpu.SemaphoreType.DMA,
            pltpu.SemaphoreType.DMA,
        ],
    )
    return k(a1, t1o, dinvo, b1, eip, zeros16)


def _tc_mm_body(x_ref, w1_ref, out_ref):
    mm = jnp.dot(x_ref[...], w1_ref[...], preferred_element_type=jnp.float32)
    out_ref[pl.ds(0, N), :] = mm
    out_ref[pl.ds(N, NP - N), :] = jnp.zeros((NP - N, HID), jnp.float32)


def _tc_mm(x, w1):
    return pl.pallas_call(
        _tc_mm_body,
        out_shape=jax.ShapeDtypeStruct((NP, HID), jnp.float32),
    )(x, w1)


def _tc_c_body(a_ref, t2_ref, dinv_ref, w2_ref, b2_ref, out_ref):
    u = dinv_ref[...] * (a_ref[0] + a_ref[1] + t2_ref[...])
    us = u[:N, :]
    z = jnp.dot(us, w2_ref[...], preferred_element_type=jnp.float32) + b2_ref[...]
    m = jnp.max(z, axis=1, keepdims=True)
    zs = z - m
    lse = jnp.log(jnp.sum(jnp.exp(zs), axis=1, keepdims=True))
    out_ref[...] = zs - lse


def _tc_c(a, t2, dinv, w2, b2):
    return pl.pallas_call(
        _tc_c_body,
        out_shape=jax.ShapeDtypeStruct((N, C), jnp.float32),
    )(a, t2, dinv, w2, b2)


def kernel(x, edge_index, W1, b1, W2, b2):
    e2 = edge_index.reshape(2, W, EPW)
    srcp = jnp.concatenate(
        [e2[0], jnp.zeros((W, PADE), jnp.int32)], axis=1).reshape(W, NCH, CH)
    dstp = jnp.concatenate(
        [e2[1], jnp.full((W, PADE), N, jnp.int32)], axis=1).reshape(W, NCH, CH)
    eip = jnp.stack([srcp, dstp])

    ones_h = jnp.ones((CH,), jnp.float32)
    zerosd = jnp.zeros((RP,), jnp.float32)
    zeros16 = jnp.zeros((RP, HID), jnp.float32)

    mm = _tc_mm(x, W1)
    a1, t1o, dinvo = _mega1(mm, eip, ones_h, zerosd, zeros16)
    a2, t2o = _mega2(a1, t1o, dinvo, b1, eip, zeros16)
    return _tc_c(a2, t2o, dinvo.reshape(NP, 1), W2, b2.reshape(1, C))
